# baseline (speedup 1.0000x reference)
import functools

import jax
import jax.numpy as jnp
from jax import lax
from jax.experimental import pallas as pl
from jax.experimental.pallas import tpu as pltpu
from jax.experimental.pallas import tpu_sc as plsc

NC = 2
NS = 16
NW = NC * NS

B = 16384
W = 200
D = 16
TT = 8
NU = W * (D // TT)
CB = 4096
NBC = B // CB
NG = CB // D

_mesh = plsc.VectorSubcoreMesh(core_axis_name="c", subcore_axis_name="s")


@functools.partial(
    pl.kernel,
    mesh=_mesh,
    compiler_params=pltpu.CompilerParams(needs_layout_passes=False),
    out_type=jax.ShapeDtypeStruct((W, D, B), jnp.float32),
    scratch_types=[
        pltpu.VMEM((4 * D,), jnp.float32),
        pltpu.VMEM((CB,), jnp.int32),
        pltpu.VMEM((CB,), jnp.int32),
        pltpu.VMEM((TT, CB), jnp.float32),
        pltpu.VMEM((TT, CB), jnp.float32),
        pltpu.SemaphoreType.DMA,
        pltpu.SemaphoreType.DMA,
        pltpu.SemaphoreType.DMA,
        pltpu.SemaphoreType.DMA,
    ],
)
def _emb_expand(idx_hbm, table_hbm, out_hbm,
                table_v, idx0, idx1, stag0, stag1,
                sin0, sin1, sout0, sout1):
    wid = lax.axis_index("s") * NC + lax.axis_index("c")
    ustart = jnp.where(wid < 16, wid * 13, 208 + (wid - 16) * 12)
    nsteps = jnp.where(wid < 16, 13 * NBC, 12 * NBC)

    idx_v = (idx0, idx1)
    stag_v = (stag0, stag1)
    sin = (sin0, sin1)
    sout = (sout0, sout1)

    pltpu.sync_copy(table_hbm, table_v)

    def locate(s):
        u = ustart + lax.shift_right_logical(s, 2)
        j = lax.shift_right_logical(u, 1)
        tt = lax.bitwise_and(u, 1)
        bc = lax.bitwise_and(s, 3)
        return j, tt, bc

    def start_in(s, b):
        j, tt, bc = locate(s)
        pltpu.async_copy(
            idx_hbm.at[j, pl.ds(bc * CB, CB)], idx_v[b], sin[b])

    def wait_in(b):
        pltpu.make_async_copy(
            idx_hbm.at[0, pl.ds(0, CB)], idx_v[b], sin[b]).wait()

    def start_out(s, b):
        j, tt, bc = locate(s)
        pltpu.async_copy(
            stag_v[b],
            out_hbm.at[j, pl.ds(tt * TT, TT), pl.ds(bc * CB, CB)], sout[b])

    def wait_out(b):
        pltpu.make_async_copy(
            stag_v[b],
            out_hbm.at[0, pl.ds(0, TT), pl.ds(0, CB)], sout[b]).wait()

    def compute(s, b):
        stag = stag_v[b]
        iv_ref = idx_v[b]
        _, tt, _ = locate(s)
        t0 = tt * TT

        rows = [
            [plsc.load_gather(
                table_v, [jnp.full((D,), r * D, jnp.int32) + (t0 + t)])
             for t in range(TT)]
            for r in range(4)
        ]

        @plsc.parallel_loop(0, NG, unroll=2)
        def group(bg):
            iv = iv_ref[pl.ds(bg * D, D)]
            m0 = lax.bitwise_and(iv, 1) != 0
            m1 = lax.bitwise_and(iv, 2) != 0
            for t in range(TT):
                lo = jnp.where(m0, rows[1][t], rows[0][t])
                hi = jnp.where(m0, rows[3][t], rows[2][t])
                stag[t, pl.ds(bg * D, D)] = jnp.where(m1, hi, lo)

    def step(s, b, first):
        wait_in(b)

        @pl.when(s + 1 < nsteps)
        def _():
            start_in(s + 1, 1 - b)

        if first is None:
            @pl.when(s >= 2)
            def _():
                wait_out(b)
        elif not first:
            wait_out(b)

        compute(s, b)
        start_out(s, b)

    start_in(jnp.int32(0), 0)
    step(jnp.int32(0), 0, True)
    step(jnp.int32(1), 1, True)

    def pair(it, carry):
        for b in range(2):
            step(it * 2 + b, b, False)
        return carry

    lax.fori_loop(1, nsteps // 2, pair, 0)

    wait_out(0)
    wait_out(1)


def kernel(indices, table):
    idx_t = indices.T
    flat_tab = table.reshape(4 * D)
    out = _emb_expand(idx_t, flat_tab)
    return out.transpose(2, 0, 1)


# speedup vs baseline: 1.0696x; 1.0696x over previous
import functools

import jax
import jax.numpy as jnp
from jax import lax
from jax.experimental import pallas as pl
from jax.experimental.pallas import tpu as pltpu
from jax.experimental.pallas import tpu_sc as plsc

NC = 2
NS = 16
NW = NC * NS

B = 16384
W = 200
D = 16
CB = 2048
NBC = B // CB
NG = CB // D
NSTEPS = W * NBC // NW

_mesh = plsc.VectorSubcoreMesh(core_axis_name="c", subcore_axis_name="s")


@functools.partial(
    pl.kernel,
    mesh=_mesh,
    compiler_params=pltpu.CompilerParams(needs_layout_passes=False),
    out_type=jax.ShapeDtypeStruct((W, D, B), jnp.float32),
    scratch_types=[
        pltpu.VMEM((4 * D,), jnp.float32),
        pltpu.VMEM((CB,), jnp.int32),
        pltpu.VMEM((CB,), jnp.int32),
        pltpu.VMEM((D, CB), jnp.float32),
        pltpu.VMEM((D, CB), jnp.float32),
        pltpu.SemaphoreType.DMA,
        pltpu.SemaphoreType.DMA,
        pltpu.SemaphoreType.DMA,
        pltpu.SemaphoreType.DMA,
    ],
)
def _emb_expand(idx_hbm, table_hbm, out_hbm,
                table_v, idx0, idx1, stag0, stag1,
                sin0, sin1, sout0, sout1):
    wid = lax.axis_index("s") * NC + lax.axis_index("c")
    gstart = wid * NSTEPS

    idx_v = (idx0, idx1)
    stag_v = (stag0, stag1)
    sin = (sin0, sin1)
    sout = (sout0, sout1)

    pltpu.sync_copy(table_hbm, table_v)

    def locate(s):
        g = gstart + s
        j = lax.shift_right_logical(g, 3)
        bc = lax.bitwise_and(g, NBC - 1)
        return j, bc

    def start_in(s, b):
        j, bc = locate(s)
        pltpu.async_copy(
            idx_hbm.at[j, pl.ds(bc * CB, CB)], idx_v[b], sin[b])

    def wait_in(b):
        pltpu.make_async_copy(
            idx_hbm.at[0, pl.ds(0, CB)], idx_v[b], sin[b]).wait()

    def start_out(s, b):
        j, bc = locate(s)
        pltpu.async_copy(
            stag_v[b], out_hbm.at[j, :, pl.ds(bc * CB, CB)], sout[b])

    def wait_out(b):
        pltpu.make_async_copy(
            stag_v[b], out_hbm.at[0, :, pl.ds(0, CB)], sout[b]).wait()

    def compute(b):
        stag = stag_v[b]
        iv_ref = idx_v[b]

        for half in range(2):
            t0 = half * (D // 2)
            rows = [
                [plsc.load_gather(
                    table_v, [jnp.full((D,), r * D + t0 + t, jnp.int32)])
                 for t in range(D // 2)]
                for r in range(4)
            ]

            @plsc.parallel_loop(0, NG, unroll=2)
            def group(bg):
                iv = iv_ref[pl.ds(bg * D, D)]
                m0 = lax.bitwise_and(iv, 1) != 0
                m1 = lax.bitwise_and(iv, 2) != 0
                for t in range(D // 2):
                    lo = jnp.where(m0, rows[1][t], rows[0][t])
                    hi = jnp.where(m0, rows[3][t], rows[2][t])
                    stag[t0 + t, pl.ds(bg * D, D)] = jnp.where(m1, hi, lo)

    def step(s, b, first):
        wait_in(b)

        @pl.when(s + 1 < NSTEPS)
        def _():
            start_in(s + 1, 1 - b)

        if not first:
            wait_out(b)

        compute(b)
        start_out(s, b)

    start_in(jnp.int32(0), 0)
    step(jnp.int32(0), 0, True)
    step(jnp.int32(1), 1, True)

    def pair(it, carry):
        for b in range(2):
            step(it * 2 + b, b, False)
        return carry

    lax.fori_loop(1, NSTEPS // 2, pair, 0)

    wait_out(0)
    wait_out(1)


def kernel(indices, table):
    idx_t = indices.T
    flat_tab = table.reshape(4 * D)
    out = _emb_expand(idx_t, flat_tab)
    return out.transpose(2, 0, 1)
